# Initial kernel scaffold; baseline (speedup 1.0000x reference)
import jax
import jax.numpy as jnp
from jax.experimental import pallas as pl


def kernel(output):
    raise NotImplementedError("write your pallas kernel here")


# speedup vs baseline: 54.6972x; 54.6972x over previous
import functools

import jax
import jax.numpy as jnp
from jax import lax
from jax.experimental import pallas as pl
from jax.experimental.pallas import tpu as pltpu
from jax.experimental.pallas import tpu_sc as plsc

_ROWS = 1024
_COLS = 32768
_LANES = 16
_NC = 2
_NS = 16
_NW = _NC * _NS
_RPW = _ROWS // _NW
_CHUNKS = _COLS // _LANES


def _row_argmax(row_buf, parity, lanes):
    m0 = jnp.full((_LANES,), -jnp.inf, jnp.float32)
    b0 = jnp.zeros((_LANES,), jnp.int32)

    def chunk_body(j, carry):
        m, bj = carry
        v = row_buf[parity, pl.ds(j * _LANES, _LANES)]
        take = v > m
        m = jnp.where(take, v, m)
        bj = jnp.where(take, jnp.full((_LANES,), j, jnp.int32), bj)
        return m, bj

    m, bj = lax.fori_loop(0, _CHUNKS, chunk_body, (m0, b0))
    col = bj * _LANES + lanes
    gm = jnp.max(m)
    cand = jnp.where(m == gm, col, jnp.int32(2**30))
    return jnp.min(cand).astype(jnp.float32)


def _argmax_rows_sc(x_flat):
    mesh = plsc.VectorSubcoreMesh(
        core_axis_name="c", subcore_axis_name="s",
        num_cores=_NC, num_subcores=_NS)

    @functools.partial(
        pl.kernel,
        out_type=jax.ShapeDtypeStruct((_ROWS,), jnp.float32),
        mesh=mesh,
        scratch_types=[
            pltpu.VMEM((2, _COLS), jnp.float32),
            pltpu.VMEM((_RPW,), jnp.float32),
            pltpu.SemaphoreType.DMA,
            pltpu.SemaphoreType.DMA,
        ],
        compiler_params=pltpu.CompilerParams(needs_layout_passes=False),
    )
    def k(x_hbm, out_hbm, row_buf, out_buf, sem0, sem1):
        wid = lax.axis_index("s") * _NC + lax.axis_index("c")
        base = wid * _RPW
        sems = (sem0, sem1)

        lanes = lax.iota(jnp.int32, _LANES)
        pending = pltpu.async_copy(x_hbm.at[base], row_buf.at[0], sems[0])
        res = jnp.zeros((_LANES,), jnp.float32)
        for r in range(_RPW):
            nxt = None
            if r + 1 < _RPW:
                nxt = pltpu.async_copy(
                    x_hbm.at[base + (r + 1)],
                    row_buf.at[(r + 1) % 2], sems[(r + 1) % 2])
            pending.wait()
            val = _row_argmax(row_buf, r % 2, lanes)
            res = jnp.where(lanes == (r % _LANES), val, res)
            if (r + 1) % _LANES == 0:
                out_buf[pl.ds((r // _LANES) * _LANES, _LANES)] = res
                res = jnp.zeros((_LANES,), jnp.float32)
            pending = nxt
        pltpu.sync_copy(out_buf, out_hbm.at[pl.ds(base, _RPW)])

    return k(x_flat)


def kernel(output):
    flat = output.reshape(_ROWS, _COLS)
    idx = _argmax_rows_sc(flat)
    return idx.reshape(64, 16, 1)


# speedup vs baseline: 189.5118x; 3.4647x over previous
import functools

import jax
import jax.numpy as jnp
from jax import lax
from jax.experimental import pallas as pl
from jax.experimental.pallas import tpu as pltpu
from jax.experimental.pallas import tpu_sc as plsc

_ROWS = 1024
_COLS = 32768
_LANES = 16
_NC = 2
_NS = 16
_NW = _NC * _NS
_RPW = _ROWS // _NW
_CHUNKS = _COLS // _LANES


def _row_argmax(row_buf, parity, lanes):
    n_acc = 4
    group = 8
    m0 = [jnp.full((_LANES,), -jnp.inf, jnp.float32) for _ in range(n_acc)]
    b0 = [jnp.zeros((_LANES,), jnp.int32) for _ in range(n_acc)]

    @plsc.parallel_loop(0, _CHUNKS, step=group, carry=(m0, b0))
    def carry_out(i, carry):
        ms, bs = carry
        ms, bs = list(ms), list(bs)
        for k in range(group):
            a = k % n_acc
            v = row_buf[parity, pl.ds((i + k) * _LANES, _LANES)]
            take = v > ms[a]
            ms[a] = jnp.where(take, v, ms[a])
            bs[a] = jnp.where(take, jnp.full((_LANES,), i + k, jnp.int32),
                              bs[a])
        return ms, bs

    ms, bs = carry_out

    def merge(m1, b1, m2, b2):
        take = (m2 > m1) | ((m2 == m1) & (b2 < b1))
        return jnp.where(take, m2, m1), jnp.where(take, b2, b1)

    m01 = merge(ms[0], bs[0], ms[1], bs[1])
    m23 = merge(ms[2], bs[2], ms[3], bs[3])
    m, bj = merge(*m01, *m23)
    col = bj * _LANES + lanes
    gm = jnp.max(m)
    cand = jnp.where(m == gm, col, jnp.int32(2**30))
    return jnp.min(cand).astype(jnp.float32)


def _argmax_rows_sc(x_flat):
    mesh = plsc.VectorSubcoreMesh(
        core_axis_name="c", subcore_axis_name="s",
        num_cores=_NC, num_subcores=_NS)

    @functools.partial(
        pl.kernel,
        out_type=jax.ShapeDtypeStruct((_ROWS,), jnp.float32),
        mesh=mesh,
        scratch_types=[
            pltpu.VMEM((2, _COLS), jnp.float32),
            pltpu.VMEM((_RPW,), jnp.float32),
            pltpu.SemaphoreType.DMA,
            pltpu.SemaphoreType.DMA,
        ],
        compiler_params=pltpu.CompilerParams(needs_layout_passes=False),
    )
    def k(x_hbm, out_hbm, row_buf, out_buf, sem0, sem1):
        wid = lax.axis_index("s") * _NC + lax.axis_index("c")
        base = wid * _RPW
        sems = (sem0, sem1)

        lanes = lax.iota(jnp.int32, _LANES)
        pending = pltpu.async_copy(x_hbm.at[base], row_buf.at[0], sems[0])
        res = jnp.zeros((_LANES,), jnp.float32)
        for r in range(_RPW):
            nxt = None
            if r + 1 < _RPW:
                nxt = pltpu.async_copy(
                    x_hbm.at[base + (r + 1)],
                    row_buf.at[(r + 1) % 2], sems[(r + 1) % 2])
            pending.wait()
            val = _row_argmax(row_buf, r % 2, lanes)
            res = jnp.where(lanes == (r % _LANES), val, res)
            if (r + 1) % _LANES == 0:
                out_buf[pl.ds((r // _LANES) * _LANES, _LANES)] = res
                res = jnp.zeros((_LANES,), jnp.float32)
            pending = nxt
        pltpu.sync_copy(out_buf, out_hbm.at[pl.ds(base, _RPW)])

    return k(x_flat)


def kernel(output):
    flat = output.reshape(_ROWS, _COLS)
    idx = _argmax_rows_sc(flat)
    return idx.reshape(64, 16, 1)
